# baseline (speedup 1.0000x reference)
import functools

import jax
import jax.numpy as jnp
from jax.experimental import pallas as pl
from jax.experimental.pallas import tpu as pltpu
from jax.experimental.pallas import tpu_sc as plsc

_WINDOW = 512


@functools.partial(jax.jit, static_argnums=())
def _sc_gather2(W_real, W_imag, idx2d):
    B = idx2d.shape[1]
    D = W_real.shape[1]
    mesh = plsc.VectorSubcoreMesh(core_axis_name="c", subcore_axis_name="s")

    NW = B // _WINDOW

    @functools.partial(
        pl.kernel,
        out_type=[
            jax.ShapeDtypeStruct((NW, _WINDOW, D), jnp.float32),
            jax.ShapeDtypeStruct((NW, _WINDOW, D), jnp.float32),
        ],
        mesh=mesh,
        compiler_params=pltpu.CompilerParams(use_tc_tiling_on_sc=False),
    )
    def k(wr_hbm, wi_hbm, idx_hbm, r_hbm, i_hbm):
        def body(idx_v, r_v, i_v):
            pltpu.sync_copy(wr_hbm.at[idx_v.at[0]], r_v.at[0])
            pltpu.sync_copy(wi_hbm.at[idx_v.at[0]], i_v.at[0])

        pltpu.emit_pipeline(
            body,
            grid=(NW,),
            in_specs=[
                pl.BlockSpec((1, _WINDOW), index_map=lambda w: (0, w)),
            ],
            out_specs=[
                pl.BlockSpec((1, _WINDOW, D), index_map=lambda w: (w, 0, 0)),
                pl.BlockSpec((1, _WINDOW, D), index_map=lambda w: (w, 0, 0)),
            ],
            core_axis_name=("c", "s"),
            dimension_semantics=(pltpu.PARALLEL,),
        )(idx_hbm, r_hbm, i_hbm)

    return k(W_real, W_imag, idx2d)


def kernel(input, W_real, W_imag):
    BATCH, HIST = input.shape
    D = W_real.shape[1]
    idx2d = input.reshape(1, BATCH * HIST)
    r, i = _sc_gather2(W_real, W_imag, idx2d)
    r_flat, i_flat = jax.lax.optimization_barrier((r.reshape(-1), i.reshape(-1)))
    out = jax.lax.optimization_barrier(jax.lax.complex(r_flat, i_flat))
    return out.reshape(BATCH, HIST, D)


# speedup vs baseline: 2.8927x; 2.8927x over previous
import functools

import jax
import jax.numpy as jnp
from jax.experimental import pallas as pl
from jax.experimental.pallas import tpu as pltpu
from jax.experimental.pallas import tpu_sc as plsc

_WINDOW = 512
_BBLK = 256


@functools.partial(jax.jit, static_argnums=())
def _sc_gather2(W_real, W_imag, idx2d):
    B = idx2d.shape[1]
    D = W_real.shape[1]
    mesh = plsc.VectorSubcoreMesh(core_axis_name="c", subcore_axis_name="s")
    NW = B // _WINDOW

    @functools.partial(
        pl.kernel,
        out_type=[
            jax.ShapeDtypeStruct((NW, _WINDOW, D), jnp.float32),
            jax.ShapeDtypeStruct((NW, _WINDOW, D), jnp.float32),
        ],
        mesh=mesh,
        compiler_params=pltpu.CompilerParams(use_tc_tiling_on_sc=False),
    )
    def k(wr_hbm, wi_hbm, idx_hbm, r_hbm, i_hbm):
        def body(idx_v, r_v, i_v):
            pltpu.sync_copy(wr_hbm.at[idx_v.at[0]], r_v.at[0])
            pltpu.sync_copy(wi_hbm.at[idx_v.at[0]], i_v.at[0])

        pltpu.emit_pipeline(
            body,
            grid=(NW,),
            in_specs=[
                pl.BlockSpec((1, _WINDOW), index_map=lambda w: (0, w)),
            ],
            out_specs=[
                pl.BlockSpec((1, _WINDOW, D), index_map=lambda w: (w, 0, 0)),
                pl.BlockSpec((1, _WINDOW, D), index_map=lambda w: (w, 0, 0)),
            ],
            core_axis_name=("c", "s"),
            dimension_semantics=(pltpu.PARALLEL,),
        )(idx_hbm, r_hbm, i_hbm)

    return k(W_real, W_imag, idx2d)


def _tc_transpose2(r2d, i2d, BATCH, HIST, D):
    def body(r_ref, i_ref, rt_ref, it_ref):
        x = r_ref[...]
        y = i_ref[...]
        for h in range(HIST):
            rt_ref[h] = x[:, h * D:(h + 1) * D].T
            it_ref[h] = y[:, h * D:(h + 1) * D].T

    out_sds = jax.ShapeDtypeStruct((HIST, D, BATCH), jnp.float32)
    return pl.pallas_call(
        body,
        grid=(BATCH // _BBLK,),
        in_specs=[
            pl.BlockSpec((_BBLK, HIST * D), lambda w: (w, 0)),
            pl.BlockSpec((_BBLK, HIST * D), lambda w: (w, 0)),
        ],
        out_specs=[
            pl.BlockSpec((HIST, D, _BBLK), lambda w: (0, 0, w)),
            pl.BlockSpec((HIST, D, _BBLK), lambda w: (0, 0, w)),
        ],
        out_shape=[out_sds, out_sds],
    )(r2d, i2d)


def kernel(input, W_real, W_imag):
    BATCH, HIST = input.shape
    D = W_real.shape[1]
    idx2d = input.reshape(1, BATCH * HIST)
    r, i = _sc_gather2(W_real, W_imag, idx2d)
    r2d = r.reshape(BATCH, HIST * D)
    i2d = i.reshape(BATCH, HIST * D)
    r_t, i_t = _tc_transpose2(r2d, i2d, BATCH, HIST, D)
    out_t = jax.lax.complex(r_t, i_t)
    return jnp.transpose(out_t, (2, 0, 1))
